# baseline (speedup 1.0000x reference)
import functools

import jax
import jax.numpy as jnp
from jax import lax
from jax.experimental import pallas as pl
from jax.experimental.pallas import tpu as pltpu
from jax.experimental.pallas import tpu_sc as plsc

_NF = 26
_VR = 100001
_DE = 32


def _gather_kernel(R, D, n_workers):
    rows_per_w = R // n_workers
    C = _NF * 64
    n_chunks = rows_per_w // C
    assert rows_per_w % C == 0
    G = C // 16

    mesh = plsc.VectorSubcoreMesh(core_axis_name="c", subcore_axis_name="s")

    @functools.partial(
        pl.kernel,
        mesh=mesh,
        out_type=jax.ShapeDtypeStruct((R, D), jnp.float32),
        compiler_params=pltpu.CompilerParams(use_tc_tiling_on_sc=False),
        scratch_types=[
            pltpu.VMEM((C,), jnp.int32),
            pltpu.VMEM((C,), jnp.int32),
            pltpu.VMEM((C, D), jnp.float32),
            pltpu.SemaphoreType.DMA,
        ],
    )
    def body(tab_hbm, idx_hbm, out_hbm, off_v, idx_v, rows_v, sem):
        nc = lax.axis_size("c")
        wid = lax.axis_index("s") * nc + lax.axis_index("c")
        base = wid * rows_per_w

        lane = lax.iota(jnp.int32, 16)

        def mk_off(g, carry):
            sl = pl.ds(g * 16, 16)
            off_v[sl] = ((g * 16 + lane) % _NF) * _VR
            return carry

        lax.fori_loop(0, G, mk_off, 0)

        def chunk(k, carry):
            cbase = base + k * C
            pltpu.sync_copy(idx_hbm.at[pl.ds(cbase, C)], idx_v)

            def add_off(g, c2):
                sl = pl.ds(g * 16, 16)
                idx_v[sl] = idx_v[sl] + off_v[sl]
                return c2

            lax.fori_loop(0, G, add_off, 0)
            pltpu.async_copy(tab_hbm.at[idx_v], rows_v, sem).wait()
            pltpu.sync_copy(rows_v, out_hbm.at[pl.ds(cbase, C)])
            return carry

        lax.fori_loop(0, n_chunks, chunk, 0)

    return body


def kernel(x, tables):
    B, F = x.shape
    Ft, V, D = tables.shape
    R = B * F
    tab_flat = tables.reshape(Ft * V, D)
    idx_flat = x.astype(jnp.int32).reshape(R)
    out_flat = _gather_kernel(R, D, 32)(tab_flat, idx_flat)
    return out_flat.reshape(B, F, D)
